# baseline (speedup 1.0000x reference)
import functools

import jax
import jax.numpy as jnp
from jax.experimental import pallas as pl
from jax.experimental.pallas import tpu as pltpu

_VOCAB = 100000
_EMBED = 128
_K = 8
_ROWS = 2048
_RB = 512
_VT = 4096
_VPAD = 102400
_NV = _VPAD // _VT
_NEG = -3.0e38


def _l2norm_kernel(x_ref, o_ref):
    x = x_ref[...]
    o_ref[...] = x / jnp.clip(jnp.sqrt(jnp.sum(x * x, axis=1, keepdims=True)),
                              1e-12)


def _topk_tile_kernel(proj_ref, table_ref, vals_ref, idx_ref,
                      sv_ref, si_ref, w_ref):
    v = pl.program_id(1)

    @pl.when(v == 0)
    def _():
        sv_ref[...] = jnp.full((_RB, _K), _NEG, jnp.float32)
        si_ref[...] = jnp.zeros((_RB, _K), jnp.int32)

    pn = proj_ref[...]
    tn = table_ref[...]

    s = jax.lax.dot_general(
        pn, tn,
        dimension_numbers=(((1,), (1,)), ((), ())),
        preferred_element_type=jnp.float32,
    )

    lane = jax.lax.broadcasted_iota(jnp.int32, (_RB, _VT), 1)
    s = jnp.where(lane + v * _VT < _VOCAB, s, _NEG)

    t8 = sv_ref[:, _K - 1:_K]
    over = s > t8
    cnt = jnp.sum(over.astype(jnp.int32), axis=1)
    rounds = jnp.minimum(jnp.max(cnt), _K)

    @pl.when(rounds > 0)
    def _():
        w_ref[...] = jnp.where(over, s, _NEG)

    big = jnp.iinfo(jnp.int32).max
    pos_col = jnp.full((_RB, 1), 3.0e38, jnp.float32)
    zero_col = jnp.zeros((_RB, 1), jnp.int32)

    def extract_round():
        w = w_ref[...]
        m = jnp.max(w, axis=1, keepdims=True)
        am = jnp.min(jnp.where(w == m, lane, big), axis=1, keepdims=True)
        w_ref[...] = jnp.where(lane == am, _NEG, w)
        widx = am + v * _VT
        cv = sv_ref[...]
        ci = si_ref[...]
        cv_sh = jnp.concatenate([pos_col, cv[:, :_K - 1]], axis=1)
        ci_sh = jnp.concatenate([zero_col, ci[:, :_K - 1]], axis=1)
        ge = cv >= m
        ge_sh = cv_sh >= m
        sv_ref[...] = jnp.where(ge, cv, jnp.where(ge_sh, m, cv_sh))
        si_ref[...] = jnp.where(ge, ci, jnp.where(ge_sh, widx, ci_sh))

    for j in range(_K):
        pl.when(rounds > j)(extract_round)

    @pl.when(v == _NV - 1)
    def _():
        vals_ref[...] = sv_ref[...]
        idx_ref[...] = si_ref[...]


@functools.partial(jax.jit, static_argnames=())
def _fused_topk(proj2d, table_padded):
    projn = pl.pallas_call(
        _l2norm_kernel,
        grid=(1,),
        in_specs=[pl.BlockSpec((_ROWS, _EMBED), lambda i: (0, 0))],
        out_specs=pl.BlockSpec((_ROWS, _EMBED), lambda i: (0, 0)),
        out_shape=jax.ShapeDtypeStruct((_ROWS, _EMBED), jnp.float32),
    )(proj2d)
    tablen = pl.pallas_call(
        _l2norm_kernel,
        grid=(_NV,),
        in_specs=[pl.BlockSpec((_VT, _EMBED), lambda i: (i, 0))],
        out_specs=pl.BlockSpec((_VT, _EMBED), lambda i: (i, 0)),
        out_shape=jax.ShapeDtypeStruct((_VPAD, _EMBED), jnp.float32),
    )(table_padded)

    grid = (_ROWS // _RB, _NV)
    vals, idx = pl.pallas_call(
        _topk_tile_kernel,
        grid=grid,
        in_specs=[
            pl.BlockSpec((_RB, _EMBED), lambda r, v: (r, 0)),
            pl.BlockSpec((_VT, _EMBED), lambda r, v: (v, 0)),
        ],
        out_specs=[
            pl.BlockSpec((_RB, _K), lambda r, v: (r, 0)),
            pl.BlockSpec((_RB, _K), lambda r, v: (r, 0)),
        ],
        out_shape=[
            jax.ShapeDtypeStruct((_ROWS, _K), jnp.float32),
            jax.ShapeDtypeStruct((_ROWS, _K), jnp.int32),
        ],
        scratch_shapes=[
            pltpu.VMEM((_RB, _K), jnp.float32),
            pltpu.VMEM((_RB, _K), jnp.int32),
            pltpu.VMEM((_RB, _VT), jnp.float32),
        ],
        compiler_params=pltpu.CompilerParams(
            dimension_semantics=("parallel", "arbitrary")),
    )(projn, tablen)
    return vals, idx


def kernel(projections, table, top_k):
    bsz, seq_len, _ = projections.shape
    proj2d = projections.reshape(_ROWS, _EMBED)
    table_padded = jnp.pad(table, ((0, _VPAD - _VOCAB), (0, 0)))

    topk_values, topk_indices = _fused_topk(proj2d, table_padded)

    probs = jax.nn.softmax(topk_values / 1.0, axis=-1)
    skey = jax.random.fold_in(jax.random.key(0), 123)
    chosen = jax.random.categorical(skey, jnp.log(probs + 1e-12), axis=-1)
    final = jnp.take_along_axis(topk_indices, chosen[:, None], axis=1)
    return final.reshape(bsz, seq_len)


# speedup vs baseline: 1.1386x; 1.1386x over previous
import functools

import jax
import jax.numpy as jnp
from jax.experimental import pallas as pl
from jax.experimental.pallas import tpu as pltpu

_VOCAB = 100000
_EMBED = 128
_K = 8
_ROWS = 2048
_RB = 512
_VT = 2048
_VPAD = 100352
_NV = _VPAD // _VT
_NEG = -3.0e38


def _l2norm_kernel(x_ref, o_ref):
    x = x_ref[...]
    o_ref[...] = x / jnp.clip(jnp.sqrt(jnp.sum(x * x, axis=1, keepdims=True)),
                              1e-12)


def _topk_tile_kernel(proj_ref, table_ref, vals_ref, idx_ref,
                      sv_ref, si_ref, w_ref):
    v = pl.program_id(1)

    @pl.when(v == 0)
    def _():
        sv_ref[...] = jnp.full((_RB, _K), _NEG, jnp.float32)
        si_ref[...] = jnp.zeros((_RB, _K), jnp.int32)

    pn = proj_ref[...]
    tn = table_ref[...]

    s = jax.lax.dot_general(
        pn, tn,
        dimension_numbers=(((1,), (1,)), ((), ())),
        preferred_element_type=jnp.float32,
    )

    lane = jax.lax.broadcasted_iota(jnp.int32, (_RB, _VT), 1)
    s = jnp.where(lane + v * _VT < _VOCAB, s, _NEG)

    t8 = sv_ref[:, _K - 1:_K]
    over = s > t8
    cnt = jnp.sum(over.astype(jnp.int32), axis=1)
    rounds = jnp.minimum(jnp.max(cnt), _K)

    @pl.when(rounds > 0)
    def _():
        w_ref[...] = s

    big = jnp.iinfo(jnp.int32).max
    pos_col = jnp.full((_RB, 1), 3.0e38, jnp.float32)
    zero_col = jnp.zeros((_RB, 1), jnp.int32)

    def extract_round():
        w = w_ref[...]
        m = jnp.max(w, axis=1, keepdims=True)
        am = jnp.min(jnp.where(w == m, lane, big), axis=1, keepdims=True)
        w_ref[...] = jnp.where(lane == am, _NEG, w)
        widx = am + v * _VT
        cv = sv_ref[...]
        ci = si_ref[...]
        cv_sh = jnp.concatenate([pos_col, cv[:, :_K - 1]], axis=1)
        ci_sh = jnp.concatenate([zero_col, ci[:, :_K - 1]], axis=1)
        ge = cv >= m
        ge_sh = cv_sh >= m
        sv_ref[...] = jnp.where(ge, cv, jnp.where(ge_sh, m, cv_sh))
        si_ref[...] = jnp.where(ge, ci, jnp.where(ge_sh, widx, ci_sh))

    for j in range(_K):
        pl.when(rounds > j)(extract_round)

    @pl.when(v == _NV - 1)
    def _():
        vals_ref[...] = sv_ref[...]
        idx_ref[...] = si_ref[...]


@functools.partial(jax.jit, static_argnames=())
def _fused_topk(proj2d, table_padded):
    projn = pl.pallas_call(
        _l2norm_kernel,
        grid=(1,),
        in_specs=[pl.BlockSpec((_ROWS, _EMBED), lambda i: (0, 0))],
        out_specs=pl.BlockSpec((_ROWS, _EMBED), lambda i: (0, 0)),
        out_shape=jax.ShapeDtypeStruct((_ROWS, _EMBED), jnp.float32),
    )(proj2d)
    tablen = pl.pallas_call(
        _l2norm_kernel,
        grid=(_NV,),
        in_specs=[pl.BlockSpec((_VT, _EMBED), lambda i: (i, 0))],
        out_specs=pl.BlockSpec((_VT, _EMBED), lambda i: (i, 0)),
        out_shape=jax.ShapeDtypeStruct((_VPAD, _EMBED), jnp.float32),
    )(table_padded)

    grid = (_ROWS // _RB, _NV)
    vals, idx = pl.pallas_call(
        _topk_tile_kernel,
        grid=grid,
        in_specs=[
            pl.BlockSpec((_RB, _EMBED), lambda r, v: (r, 0)),
            pl.BlockSpec((_VT, _EMBED), lambda r, v: (v, 0)),
        ],
        out_specs=[
            pl.BlockSpec((_RB, _K), lambda r, v: (r, 0)),
            pl.BlockSpec((_RB, _K), lambda r, v: (r, 0)),
        ],
        out_shape=[
            jax.ShapeDtypeStruct((_ROWS, _K), jnp.float32),
            jax.ShapeDtypeStruct((_ROWS, _K), jnp.int32),
        ],
        scratch_shapes=[
            pltpu.VMEM((_RB, _K), jnp.float32),
            pltpu.VMEM((_RB, _K), jnp.int32),
            pltpu.VMEM((_RB, _VT), jnp.float32),
        ],
        compiler_params=pltpu.CompilerParams(
            dimension_semantics=("parallel", "arbitrary")),
    )(projn, tablen)
    return vals, idx


def kernel(projections, table, top_k):
    bsz, seq_len, _ = projections.shape
    proj2d = projections.reshape(_ROWS, _EMBED)
    table_padded = jnp.pad(table, ((0, _VPAD - _VOCAB), (0, 0)))

    topk_values, topk_indices = _fused_topk(proj2d, table_padded)

    probs = jax.nn.softmax(topk_values / 1.0, axis=-1)
    skey = jax.random.fold_in(jax.random.key(0), 123)
    chosen = jax.random.categorical(skey, jnp.log(probs + 1e-12), axis=-1)
    final = jnp.take_along_axis(topk_indices, chosen[:, None], axis=1)
    return final.reshape(bsz, seq_len)


# speedup vs baseline: 1.1652x; 1.0234x over previous
import functools

import jax
import jax.numpy as jnp
from jax.experimental import pallas as pl
from jax.experimental.pallas import tpu as pltpu

_VOCAB = 100000
_EMBED = 128
_K = 8
_ROWS = 2048
_RB = 512
_VT = 2048
_VPAD = 100352
_NV = _VPAD // _VT
_NEG = -3.0e38


def _l2norm_kernel(x_ref, o_ref):
    x = x_ref[...]
    o_ref[...] = x / jnp.clip(jnp.sqrt(jnp.sum(x * x, axis=1, keepdims=True)),
                              1e-12)


def _topk_tile_kernel(proj_ref, table_ref, vals_ref, idx_ref,
                      sv_ref, si_ref, w_ref):
    v = pl.program_id(1)

    @pl.when(v == 0)
    def _():
        sv_ref[...] = jnp.full((_RB, _K), _NEG, jnp.float32)
        si_ref[...] = jnp.zeros((_RB, _K), jnp.int32)

    pn = proj_ref[...]
    tn = table_ref[...]

    s = jax.lax.dot_general(
        pn, tn,
        dimension_numbers=(((1,), (1,)), ((), ())),
        preferred_element_type=jnp.float32,
    )

    lane = jax.lax.broadcasted_iota(jnp.int32, (_RB, _VT), 1)
    s = jnp.where(lane + v * _VT < _VOCAB, s, _NEG)

    t8 = sv_ref[:, _K - 1:_K]
    over = s > t8
    cnt = jnp.sum(over.astype(jnp.int32), axis=1)
    rounds = jnp.minimum(jnp.max(cnt), _K)

    @pl.when(rounds > 0)
    def _():
        w_ref[...] = s

    big = jnp.iinfo(jnp.int32).max
    pos_col = jnp.full((_RB, 1), 3.0e38, jnp.float32)
    zero_col = jnp.zeros((_RB, 1), jnp.int32)

    def extract_round():
        w = w_ref[...]
        m = jnp.max(w, axis=1, keepdims=True)
        am = jnp.min(jnp.where(w == m, lane, big), axis=1, keepdims=True)
        w_ref[...] = jnp.where(lane == am, _NEG, w)
        widx = am + v * _VT
        cv = sv_ref[...]
        ci = si_ref[...]
        cv_sh = jnp.concatenate([pos_col, cv[:, :_K - 1]], axis=1)
        ci_sh = jnp.concatenate([zero_col, ci[:, :_K - 1]], axis=1)
        ge = cv >= m
        ge_sh = cv_sh >= m
        sv_ref[...] = jnp.where(ge, cv, jnp.where(ge_sh, m, cv_sh))
        si_ref[...] = jnp.where(ge, ci, jnp.where(ge_sh, widx, ci_sh))

    for j in range(_K):
        pl.when(rounds > j)(extract_round)

    @pl.when(v == _NV - 1)
    def _():
        vals_ref[...] = sv_ref[...]
        idx_ref[...] = si_ref[...]


@functools.partial(jax.jit, static_argnames=())
def _fused_topk(proj2d, table):
    projn = pl.pallas_call(
        _l2norm_kernel,
        grid=(1,),
        in_specs=[pl.BlockSpec((_ROWS, _EMBED), lambda i: (0, 0))],
        out_specs=pl.BlockSpec((_ROWS, _EMBED), lambda i: (0, 0)),
        out_shape=jax.ShapeDtypeStruct((_ROWS, _EMBED), jnp.float32),
    )(proj2d)
    tablen = pl.pallas_call(
        _l2norm_kernel,
        grid=(_NV,),
        in_specs=[pl.BlockSpec((_VT, _EMBED), lambda i: (i, 0))],
        out_specs=pl.BlockSpec((_VT, _EMBED), lambda i: (i, 0)),
        out_shape=jax.ShapeDtypeStruct((_VPAD, _EMBED), jnp.float32),
    )(table)

    grid = (_ROWS // _RB, _NV)
    vals, idx = pl.pallas_call(
        _topk_tile_kernel,
        grid=grid,
        in_specs=[
            pl.BlockSpec((_RB, _EMBED), lambda r, v: (r, 0)),
            pl.BlockSpec((_VT, _EMBED), lambda r, v: (v, 0)),
        ],
        out_specs=[
            pl.BlockSpec((_RB, _K), lambda r, v: (r, 0)),
            pl.BlockSpec((_RB, _K), lambda r, v: (r, 0)),
        ],
        out_shape=[
            jax.ShapeDtypeStruct((_ROWS, _K), jnp.float32),
            jax.ShapeDtypeStruct((_ROWS, _K), jnp.int32),
        ],
        scratch_shapes=[
            pltpu.VMEM((_RB, _K), jnp.float32),
            pltpu.VMEM((_RB, _K), jnp.int32),
            pltpu.VMEM((_RB, _VT), jnp.float32),
        ],
        compiler_params=pltpu.CompilerParams(
            dimension_semantics=("parallel", "arbitrary")),
    )(projn, tablen)
    return vals, idx


def kernel(projections, table, top_k):
    bsz, seq_len, _ = projections.shape
    proj2d = projections.reshape(_ROWS, _EMBED)
    topk_values, topk_indices = _fused_topk(proj2d, table)

    probs = jax.nn.softmax(topk_values / 1.0, axis=-1)
    skey = jax.random.fold_in(jax.random.key(0), 123)
    chosen = jax.random.categorical(skey, jnp.log(probs + 1e-12), axis=-1)
    final = jnp.take_along_axis(topk_indices, chosen[:, None], axis=1)
    return final.reshape(bsz, seq_len)
